# baseline (speedup 1.0000x reference)
import functools

import jax
import jax.numpy as jnp
from jax import lax
from jax.experimental import pallas as pl
from jax.experimental.pallas import tpu as pltpu
from jax.experimental.pallas import tpu_sc as plsc

NN = 10000
BSZ = 10
SEQ = 1000
NH = 6
CH = 512
HC = NH * CH
G = 16
E_RAW = 160000
E1 = E_RAW + NN
E_PAD = E1 + (G - 1) * NN
NW = 32

_SELU_L = 1.0507009873554805
_SELU_A = 1.6732632423543772


def _t1_body(mf_ref, x_ref, Wemb, bemb, Wq, bq, Wk, bk, Wv, bv, Wo, bo,
             g1, b1, Wff1, bff1, Wff2, bff2, g2, b2, Wtout, btout,
             Wlin, blin, hid_ref):
    f32 = jnp.float32

    def ln(t, g, b):
        mu = t.mean(-1, keepdims=True)
        var = t.var(-1, keepdims=True)
        return (t - mu) / jnp.sqrt(var + 1e-5) * g[...] + b[...]

    mf = mf_ref[0]
    h = jnp.dot(mf, Wemb[...], preferred_element_type=f32) + bemb[...]
    q = jnp.dot(h, Wq[...], preferred_element_type=f32) + bq[...]
    k = jnp.dot(h, Wk[...], preferred_element_type=f32) + bk[...]
    v = jnp.dot(h, Wv[...], preferred_element_type=f32) + bv[...]
    ao_heads = []
    for hh in range(4):
        sl = slice(hh * 16, (hh + 1) * 16)
        qs, ks, vs = q[:, sl], k[:, sl], v[:, sl]
        s = lax.dot_general(qs, ks, (((1,), (1,)), ((), ())),
                            preferred_element_type=f32) * 0.25
        s = jax.nn.softmax(s, axis=-1)
        ao_heads.append(jnp.dot(s, vs, preferred_element_type=f32))
    ao = jnp.concatenate(ao_heads, axis=-1)
    h = ln(h + jnp.dot(ao, Wo[...], preferred_element_type=f32) + bo[...],
           g1, b1)
    ff = jnp.dot(jax.nn.relu(jnp.dot(h, Wff1[...], preferred_element_type=f32)
                             + bff1[...]),
                 Wff2[...], preferred_element_type=f32) + bff2[...]
    h = ln(h + ff, g2, b2)
    feats = jnp.dot(h, Wtout[...], preferred_element_type=f32) + btout[...]
    xx = x_ref[0]
    fc = jnp.concatenate([xx[:, 2:], feats], axis=1)
    z = jnp.dot(fc, Wlin[...], preferred_element_type=f32) + blin[...]
    en = jnp.exp(jnp.minimum(z, 0.0))
    hid_ref[0] = _SELU_L * jnp.where(z > 0.0, z, _SELU_A * (en - 1.0))


def _t1(mf3, x3, Wemb, bemb, Wq, bq, Wk, bk, Wv, bv, Wo, bo, g1, b1,
        Wff1, bff1, Wff2, bff2, g2, b2, Wtout, btout, Wlin, blin):
    full = lambda *shape: pl.BlockSpec(shape, lambda b: (0,) * len(shape))
    return pl.pallas_call(
        _t1_body,
        grid=(BSZ,),
        in_specs=[
            pl.BlockSpec((1, SEQ, 4), lambda b: (b, 0, 0)),
            pl.BlockSpec((1, SEQ, 7), lambda b: (b, 0, 0)),
            full(4, 64), full(64), full(64, 64), full(64), full(64, 64),
            full(64), full(64, 64), full(64), full(64, 64), full(64),
            full(64), full(64), full(64, 256), full(256), full(256, 64),
            full(64), full(64), full(64), full(64, 16), full(16),
            full(21, 512), full(512),
        ],
        out_specs=pl.BlockSpec((1, SEQ, CH), lambda b: (b, 0, 0)),
        out_shape=jax.ShapeDtypeStruct((BSZ, SEQ, CH), jnp.float32),
    )(mf3, x3, Wemb, bemb, Wq, bq, Wk, bk, Wv, bv, Wo, bo, g1, b1,
      Wff1, bff1, Wff2, bff2, g2, b2, Wtout, btout, Wlin, blin)


_BMROW = 1000
_BCOL = 1024


def _t2_body(xin_ref, Wl_ref, bl_ref, Wr_ref, br_ref, xl_ref, xr_ref):
    f32 = jnp.float32
    h = xin_ref[...]
    xl_ref[...] = (jnp.dot(h, Wl_ref[...], preferred_element_type=f32)
                   + bl_ref[...])
    xr_ref[...] = (jnp.dot(h, Wr_ref[...], preferred_element_type=f32)
                   + br_ref[...])


def _t2(xin, W_l, bl2, W_r, br2):
    return pl.pallas_call(
        _t2_body,
        grid=(NN // _BMROW, HC // _BCOL),
        in_specs=[
            pl.BlockSpec((_BMROW, 513), lambda i, j: (i, 0)),
            pl.BlockSpec((513, _BCOL), lambda i, j: (0, j)),
            pl.BlockSpec((1, _BCOL), lambda i, j: (0, j)),
            pl.BlockSpec((513, _BCOL), lambda i, j: (0, j)),
            pl.BlockSpec((1, _BCOL), lambda i, j: (0, j)),
        ],
        out_specs=[
            pl.BlockSpec((_BMROW, _BCOL), lambda i, j: (i, j)),
            pl.BlockSpec((_BMROW, _BCOL), lambda i, j: (i, j)),
        ],
        out_shape=[
            jax.ShapeDtypeStruct((NN, HC), jnp.float32),
            jax.ShapeDtypeStruct((NN, HC), jnp.float32),
        ],
    )(xin, W_l, bl2, W_r, br2)


def _k3_body(hid_ref, w_ref, b_ref, phi_ref):
    phi_ref[...] = (jnp.dot(hid_ref[...], w_ref[...],
                            preferred_element_type=jnp.float32)
                    + b_ref[...])


def _k3(hidden, W_out2, b2d):
    return pl.pallas_call(
        _k3_body,
        grid=(NN // _BMROW,),
        in_specs=[
            pl.BlockSpec((_BMROW, CH), lambda i: (i, 0)),
            pl.BlockSpec((CH, 1), lambda i: (0, 0)),
            pl.BlockSpec((1, 1), lambda i: (0, 0)),
        ],
        out_specs=pl.BlockSpec((_BMROW, 1), lambda i: (i, 0)),
        out_shape=jax.ShapeDtypeStruct((NN, 1), jnp.float32),
    )(hidden, W_out2, b2d)


def _sc_gat_body(xl_hbm, xr_hbm, idx_hbm, meta_hbm, bounds_hbm, att_hbm,
                 gatb_hbm, hid_hbm,
                 att_v, gatb_v, xr_v, rows_v, idx_v, meta_v, bnd_v,
                 acc_v, mbuf, sbuf, hidb, sem):
    f32 = jnp.float32
    wid = lax.axis_index("s") * 2 + lax.axis_index("c")
    pltpu.sync_copy(att_hbm, att_v)
    pltpu.sync_copy(gatb_hbm, gatb_v)
    pltpu.sync_copy(bounds_hbm.at[wid], bnd_v)
    bvec = bnd_v[...]
    n0 = bvec[0]
    n1 = bvec[1]
    iota16 = lax.iota(jnp.int32, 16)

    neg_big = jnp.full((16,), -3e38, f32)

    def _stage_idx(j, slot):
        off_e = pl.multiple_of(start_ref[0] + j * G, G)
        dst = pl.multiple_of(slot * G, G)
        pltpu.sync_copy(idx_hbm.at[pl.ds(off_e, G)],
                        idx_v.at[pl.ds(dst, G)])

    def _gather(j, slot, sem):
        dst = pl.multiple_of(slot * G, G)
        return pltpu.make_async_copy(
            xl_hbm.at[idx_v.at[pl.ds(dst, G)]],
            rows_v.at[pl.ds(dst, G)], sem)

    start_ref = [None]

    def node_body(n, carry):
        pltpu.sync_copy(meta_hbm.at[n], meta_v)
        pltpu.sync_copy(xr_hbm.at[n], xr_v)
        mvec = meta_v[...]
        start = mvec[0]
        nch = mvec[1]
        deg = mvec[2]
        start_ref[0] = start
        mbuf[...] = neg_big

        _stage_idx(0, 0)
        _gather(0, 0, sem).start()

        @pl.when(nch > 1)
        def _():
            _stage_idx(1, 1)

        def chunk_body(j, c2):
            slot = j & 1
            _gather(j, slot, sem).wait()

            @pl.when(j + 1 < nch)
            def _():
                _gather(j + 1, 1 - slot, sem).start()

            @pl.when(j + 2 < nch)
            def _():
                _stage_idx(j + 2, slot)

            rbase = slot * G
            lvs = [None] * G
            for hh in range(NH):
                off0 = hh * CH

                def cb(ci, vas):
                    off = off0 + ci * 16
                    xrv = xr_v[pl.ds(off, 16)]
                    atv = att_v[pl.ds(off, 16)]
                    out = []
                    for e in range(G):
                        z = rows_v[rbase + e, pl.ds(off, 16)] + xrv
                        z = jnp.maximum(z, z * 0.2)
                        out.append(vas[e] + z * atv)
                    return tuple(out)
                vas = lax.fori_loop(0, CH // 16, cb,
                                    (jnp.zeros((16,), f32),) * G, unroll=4)
                for e in range(G):
                    lg = jnp.sum(vas[e])
                    lv = lvs[e]
                    lvs[e] = (jnp.where(iota16 == hh, lg, neg_big) if lv is None
                              else jnp.where(iota16 == hh, lg, lv))
            base_e = j * G
            for e in range(G):
                lvs[e] = jnp.where(base_e + e < deg, lvs[e], neg_big)
            def _tree(vals, op):
                vals = list(vals)
                while len(vals) > 1:
                    vals = [op(vals[i], vals[i + 1])
                            for i in range(0, len(vals), 2)]
                return vals[0]

            mc = _tree(lvs, jnp.maximum)
            m = mbuf[...]
            mn = jnp.maximum(m, mc)
            scv = jnp.exp(m - mn)
            mbuf[...] = mn
            ws = [jnp.exp(lv - mn) for lv in lvs]
            wsum = _tree(ws, jnp.add)
            sbuf[...] = sbuf[...] * scv + wsum
            sch = tuple(scv[hh] for hh in range(NH))
            weh = [[ws[e][hh] for hh in range(NH)] for e in range(G)]
            for hh in range(NH):
                off0 = hh * CH
                sc_h = sch[hh]
                w_h = [weh[e][hh] for e in range(G)]

                def ab(ci, c4):
                    off = off0 + ci * 16
                    terms = [rows_v[rbase + e, pl.ds(off, 16)] * w_h[e]
                             for e in range(G)]
                    while len(terms) > 1:
                        terms = [terms[i] + terms[i + 1]
                                 for i in range(0, len(terms), 2)]
                    acc_v[pl.ds(off, 16)] = (acc_v[pl.ds(off, 16)] * sc_h
                                             + terms[0])
                    return c4
                lax.fori_loop(0, CH // 16, ab, 0, unroll=4)
            return c2
        lax.fori_loop(0, nch, chunk_body, 0)

        coef = 1.0 / (6.0 * (sbuf[...] + 1e-16))
        cs = tuple(coef[hh] for hh in range(NH))

        def fb(ci, c):
            off = ci * 16
            hv = acc_v[pl.ds(off, 16)] * cs[0]
            for hh in range(1, NH):
                hv = hv + acc_v[pl.ds(hh * CH + off, 16)] * cs[hh]
            z = hv + gatb_v[pl.ds(off, 16)]
            en = jnp.exp(jnp.minimum(z, 0.0))
            val = jnp.where(z > 0.0, z, _SELU_A * (en - 1.0))
            hidb[pl.ds(off, 16)] = _SELU_L * val
            return c
        lax.fori_loop(0, CH // 16, fb, 0, unroll=2)
        pltpu.sync_copy(hidb, hid_hbm.at[n])
        return carry
    lax.fori_loop(n0, n1, node_body, 0)


@functools.partial(
    pl.kernel,
    out_type=jax.ShapeDtypeStruct((NN, CH), jnp.float32),
    mesh=plsc.VectorSubcoreMesh(core_axis_name="c", subcore_axis_name="s"),
    compiler_params=pltpu.CompilerParams(needs_layout_passes=False),
    scratch_types=[
        pltpu.VMEM((HC,), jnp.float32),
        pltpu.VMEM((CH,), jnp.float32),
        pltpu.VMEM((HC,), jnp.float32),
        pltpu.VMEM((2 * G, HC), jnp.float32),
        pltpu.VMEM((2 * G,), jnp.int32),
        pltpu.VMEM((16,), jnp.int32),
        pltpu.VMEM((16,), jnp.int32),
        pltpu.VMEM((HC,), jnp.float32),
        pltpu.VMEM((16,), jnp.float32),
        pltpu.VMEM((16,), jnp.float32),
        pltpu.VMEM((CH,), jnp.float32),
        pltpu.SemaphoreType.DMA,
    ],
)
def _sc_gat(xl_hbm, xr_hbm, idx_hbm, meta_hbm, bounds_hbm, att_hbm,
            gatb_hbm, hid_hbm, *scratch):
    _sc_gat_body(xl_hbm, xr_hbm, idx_hbm, meta_hbm, bounds_hbm, att_hbm,
                 gatb_hbm, hid_hbm, *scratch)


def kernel(edge_index, mesh_feat, x, conv_feat, W_emb, b_emb, Wq, bq, Wk, bk,
           Wv, bv, Wo, bo, ln1_g, ln1_b, W_ff1, b_ff1, W_ff2, b_ff2, ln2_g,
           ln2_b, W_tout, b_tout, W_lin, b_lin, W_l, b_l, W_r, b_r, att,
           gat_b, W_out2, b_out2):
    i32 = jnp.int32

    mf3 = mesh_feat.reshape(BSZ, SEQ, 4)
    x3 = x.reshape(BSZ, SEQ, 7)
    hidden = _t1(mf3, x3, W_emb, b_emb, Wq, bq, Wk, bk, Wv, bv, Wo, bo,
                 ln1_g, ln1_b, W_ff1, b_ff1, W_ff2, b_ff2, ln2_g, ln2_b,
                 W_tout, b_tout, W_lin, b_lin).reshape(NN, CH)
    phi = mesh_feat[:, 3:4]

    ar = jnp.arange(NN, dtype=edge_index.dtype)
    src_all = jnp.concatenate([edge_index[0], ar])
    dst_all = jnp.concatenate([edge_index[1], ar])
    order = jnp.argsort(dst_all)
    ssrc = src_all[order].astype(i32)
    sdst = dst_all[order].astype(i32)
    deg = jnp.zeros((NN,), i32).at[dst_all].add(1)
    pdeg = ((deg + (G - 1)) // G) * G
    prow = jnp.concatenate([jnp.zeros((1,), i32), jnp.cumsum(pdeg)])
    rrow = jnp.concatenate([jnp.zeros((1,), i32), jnp.cumsum(deg)])
    pos = prow[sdst] + (jnp.arange(E1, dtype=i32) - rrow[sdst])
    idx_pad = jnp.zeros((E_PAD,), i32).at[pos].set(ssrc)
    meta = (jnp.zeros((NN, 16), i32)
            .at[:, 0].set(prow[:NN])
            .at[:, 1].set(pdeg // G)
            .at[:, 2].set(deg))
    cumch = prow // G
    total_ch = cumch[NN]
    targets = (jnp.arange(1, NW, dtype=i32) * total_ch) // NW
    bvals = jnp.searchsorted(cumch, targets).astype(i32)
    bounds = jnp.concatenate(
        [jnp.zeros((1,), i32), bvals, jnp.full((1,), NN, i32)])
    bounds2 = (jnp.zeros((NW, 16), i32)
               .at[:, 0].set(bounds[:NW])
               .at[:, 1].set(bounds[1:]))

    attf = att.reshape(HC)
    bl2, br2 = b_l.reshape(1, HC), b_r.reshape(1, HC)
    b2d = b_out2.reshape(1, 1)

    for _ in range(3):
        xin = jnp.concatenate([phi, hidden], axis=1)
        xl, xr = _t2(xin, W_l, bl2, W_r, br2)
        hidden = _sc_gat(xl, xr, idx_pad, meta, bounds2, attf, gat_b)
        phi = _k3(hidden, W_out2, b2d)
    return phi


# speedup vs baseline: 1.5325x; 1.5325x over previous
import functools

import jax
import jax.numpy as jnp
from jax import lax
from jax.experimental import pallas as pl
from jax.experimental.pallas import tpu as pltpu
from jax.experimental.pallas import tpu_sc as plsc

NN = 10000
BSZ = 10
SEQ = 1000
NH = 6
CH = 512
HC = NH * CH
G = 8
E_RAW = 160000
E1 = E_RAW + NN
E_PAD = E1 + (G - 1) * NN
NW = 32

_SELU_L = 1.0507009873554805
_SELU_A = 1.6732632423543772


def _t1_body(mf_ref, x_ref, Wemb, bemb, Wq, bq, Wk, bk, Wv, bv, Wo, bo,
             g1, b1, Wff1, bff1, Wff2, bff2, g2, b2, Wtout, btout,
             Wlin, blin, hid_ref):
    f32 = jnp.float32

    def ln(t, g, b):
        mu = t.mean(-1, keepdims=True)
        var = t.var(-1, keepdims=True)
        return (t - mu) / jnp.sqrt(var + 1e-5) * g[...] + b[...]

    mf = mf_ref[0]
    h = jnp.dot(mf, Wemb[...], preferred_element_type=f32) + bemb[...]
    q = jnp.dot(h, Wq[...], preferred_element_type=f32) + bq[...]
    k = jnp.dot(h, Wk[...], preferred_element_type=f32) + bk[...]
    v = jnp.dot(h, Wv[...], preferred_element_type=f32) + bv[...]
    ao_heads = []
    for hh in range(4):
        sl = slice(hh * 16, (hh + 1) * 16)
        qs, ks, vs = q[:, sl], k[:, sl], v[:, sl]
        s = lax.dot_general(qs, ks, (((1,), (1,)), ((), ())),
                            preferred_element_type=f32) * 0.25
        s = jax.nn.softmax(s, axis=-1)
        ao_heads.append(jnp.dot(s, vs, preferred_element_type=f32))
    ao = jnp.concatenate(ao_heads, axis=-1)
    h = ln(h + jnp.dot(ao, Wo[...], preferred_element_type=f32) + bo[...],
           g1, b1)
    ff = jnp.dot(jax.nn.relu(jnp.dot(h, Wff1[...], preferred_element_type=f32)
                             + bff1[...]),
                 Wff2[...], preferred_element_type=f32) + bff2[...]
    h = ln(h + ff, g2, b2)
    feats = jnp.dot(h, Wtout[...], preferred_element_type=f32) + btout[...]
    xx = x_ref[0]
    fc = jnp.concatenate([xx[:, 2:], feats], axis=1)
    z = jnp.dot(fc, Wlin[...], preferred_element_type=f32) + blin[...]
    en = jnp.exp(jnp.minimum(z, 0.0))
    hid_ref[0] = _SELU_L * jnp.where(z > 0.0, z, _SELU_A * (en - 1.0))


def _t1(mf3, x3, Wemb, bemb, Wq, bq, Wk, bk, Wv, bv, Wo, bo, g1, b1,
        Wff1, bff1, Wff2, bff2, g2, b2, Wtout, btout, Wlin, blin):
    full = lambda *shape: pl.BlockSpec(shape, lambda b: (0,) * len(shape))
    return pl.pallas_call(
        _t1_body,
        grid=(BSZ,),
        in_specs=[
            pl.BlockSpec((1, SEQ, 4), lambda b: (b, 0, 0)),
            pl.BlockSpec((1, SEQ, 7), lambda b: (b, 0, 0)),
            full(4, 64), full(64), full(64, 64), full(64), full(64, 64),
            full(64), full(64, 64), full(64), full(64, 64), full(64),
            full(64), full(64), full(64, 256), full(256), full(256, 64),
            full(64), full(64), full(64), full(64, 16), full(16),
            full(21, 512), full(512),
        ],
        out_specs=pl.BlockSpec((1, SEQ, CH), lambda b: (b, 0, 0)),
        out_shape=jax.ShapeDtypeStruct((BSZ, SEQ, CH), jnp.float32),
    )(mf3, x3, Wemb, bemb, Wq, bq, Wk, bk, Wv, bv, Wo, bo, g1, b1,
      Wff1, bff1, Wff2, bff2, g2, b2, Wtout, btout, Wlin, blin)


_BMROW = 1000
_BCOL = 1024


def _t2_body(xin_ref, Wl_ref, bl_ref, Wr_ref, br_ref, xl_ref, xr_ref):
    f32 = jnp.float32
    h = xin_ref[...]
    xl_ref[...] = (jnp.dot(h, Wl_ref[...], preferred_element_type=f32)
                   + bl_ref[...])
    xr_ref[...] = (jnp.dot(h, Wr_ref[...], preferred_element_type=f32)
                   + br_ref[...])


def _t2(xin, W_l, bl2, W_r, br2):
    return pl.pallas_call(
        _t2_body,
        grid=(NN // _BMROW, HC // _BCOL),
        in_specs=[
            pl.BlockSpec((_BMROW, 513), lambda i, j: (i, 0)),
            pl.BlockSpec((513, _BCOL), lambda i, j: (0, j)),
            pl.BlockSpec((1, _BCOL), lambda i, j: (0, j)),
            pl.BlockSpec((513, _BCOL), lambda i, j: (0, j)),
            pl.BlockSpec((1, _BCOL), lambda i, j: (0, j)),
        ],
        out_specs=[
            pl.BlockSpec((_BMROW, _BCOL), lambda i, j: (i, j)),
            pl.BlockSpec((_BMROW, _BCOL), lambda i, j: (i, j)),
        ],
        out_shape=[
            jax.ShapeDtypeStruct((NN, HC), jnp.float32),
            jax.ShapeDtypeStruct((NN, HC), jnp.float32),
        ],
    )(xin, W_l, bl2, W_r, br2)


def _k3_body(hid_ref, w_ref, b_ref, phi_ref):
    phi_ref[...] = (jnp.dot(hid_ref[...], w_ref[...],
                            preferred_element_type=jnp.float32)
                    + b_ref[...])


def _k3(hidden, W_out2, b2d):
    return pl.pallas_call(
        _k3_body,
        grid=(NN // _BMROW,),
        in_specs=[
            pl.BlockSpec((_BMROW, CH), lambda i: (i, 0)),
            pl.BlockSpec((CH, 1), lambda i: (0, 0)),
            pl.BlockSpec((1, 1), lambda i: (0, 0)),
        ],
        out_specs=pl.BlockSpec((_BMROW, 1), lambda i: (i, 0)),
        out_shape=jax.ShapeDtypeStruct((NN, 1), jnp.float32),
    )(hidden, W_out2, b2d)


def _sc_gat_body(xl_hbm, xr_hbm, idx_hbm, meta_hbm, bounds_hbm, att_hbm,
                 gatb_hbm, hid_hbm,
                 att_v, gatb_v, xr_v, rows_v, idx_v, meta_v, bnd_v,
                 acc_v, mbuf, sbuf, hidb, sem):
    f32 = jnp.float32
    wid = lax.axis_index("s") * 2 + lax.axis_index("c")
    pltpu.sync_copy(att_hbm, att_v)
    pltpu.sync_copy(gatb_hbm, gatb_v)
    pltpu.sync_copy(bounds_hbm.at[wid], bnd_v)
    bvec = bnd_v[...]
    n0 = bvec[0]
    n1 = bvec[1]
    iota16 = lax.iota(jnp.int32, 16)

    neg_big = jnp.full((16,), -3e38, f32)

    def _stage_idx(j, slot):
        off_e = pl.multiple_of(start_ref[0] + j * G, G)
        dst = pl.multiple_of(slot * G, G)
        pltpu.sync_copy(idx_hbm.at[pl.ds(off_e, G)],
                        idx_v.at[pl.ds(dst, G)])

    def _gather(j, slot, sem):
        dst = pl.multiple_of(slot * G, G)
        return pltpu.make_async_copy(
            xl_hbm.at[idx_v.at[pl.ds(dst, G)]],
            rows_v.at[pl.ds(dst, G)], sem)

    start_ref = [None]

    def node_body(n, carry):
        pltpu.sync_copy(meta_hbm.at[n], meta_v)
        pltpu.sync_copy(xr_hbm.at[n], xr_v)
        mvec = meta_v[...]
        start = mvec[0]
        nch = mvec[1]
        deg = mvec[2]
        start_ref[0] = start
        mbuf[...] = neg_big

        _stage_idx(0, 0)
        _gather(0, 0, sem).start()

        @pl.when(nch > 1)
        def _():
            _stage_idx(1, 1)

        def chunk_body(j, c2):
            slot = j & 1
            _gather(j, slot, sem).wait()

            @pl.when(j + 1 < nch)
            def _():
                _gather(j + 1, 1 - slot, sem).start()

            @pl.when(j + 2 < nch)
            def _():
                _stage_idx(j + 2, slot)

            rbase = slot * G
            lvs = [None] * G
            for hh in range(NH):
                off0 = hh * CH

                def cb(ci, vas):
                    off = off0 + ci * 16
                    xrv = xr_v[pl.ds(off, 16)]
                    atv = att_v[pl.ds(off, 16)]
                    out = []
                    for e in range(G):
                        z = rows_v[rbase + e, pl.ds(off, 16)] + xrv
                        z = jnp.maximum(z, z * 0.2)
                        out.append(vas[e] + z * atv)
                    return tuple(out)
                vas = lax.fori_loop(0, CH // 16, cb,
                                    (jnp.zeros((16,), f32),) * G, unroll=4)
                for e in range(G):
                    lg = jnp.sum(vas[e])
                    lv = lvs[e]
                    lvs[e] = (jnp.where(iota16 == hh, lg, neg_big) if lv is None
                              else jnp.where(iota16 == hh, lg, lv))
            base_e = j * G
            for e in range(G):
                lvs[e] = jnp.where(base_e + e < deg, lvs[e], neg_big)
            def _tree(vals, op):
                vals = list(vals)
                while len(vals) > 1:
                    vals = [op(vals[i], vals[i + 1])
                            for i in range(0, len(vals), 2)]
                return vals[0]

            mc = _tree(lvs, jnp.maximum)
            m = mbuf[...]
            mn = jnp.maximum(m, mc)
            scv = jnp.exp(m - mn)
            mbuf[...] = mn
            ws = [jnp.exp(lv - mn) for lv in lvs]
            wsum = _tree(ws, jnp.add)
            sbuf[...] = sbuf[...] * scv + wsum
            sch = tuple(scv[hh] for hh in range(NH))
            weh = [[ws[e][hh] for hh in range(NH)] for e in range(G)]
            for hh in range(NH):
                off0 = hh * CH
                sc_h = sch[hh]
                w_h = [weh[e][hh] for e in range(G)]

                def ab(ci, c4):
                    off = off0 + ci * 16
                    terms = [rows_v[rbase + e, pl.ds(off, 16)] * w_h[e]
                             for e in range(G)]
                    while len(terms) > 1:
                        terms = [terms[i] + terms[i + 1]
                                 for i in range(0, len(terms), 2)]
                    acc_v[pl.ds(off, 16)] = (acc_v[pl.ds(off, 16)] * sc_h
                                             + terms[0])
                    return c4
                lax.fori_loop(0, CH // 16, ab, 0, unroll=4)
            return c2
        lax.fori_loop(0, nch, chunk_body, 0)

        coef = 1.0 / (6.0 * (sbuf[...] + 1e-16))
        cs = tuple(coef[hh] for hh in range(NH))

        def fb(ci, c):
            off = ci * 16
            hv = acc_v[pl.ds(off, 16)] * cs[0]
            for hh in range(1, NH):
                hv = hv + acc_v[pl.ds(hh * CH + off, 16)] * cs[hh]
            z = hv + gatb_v[pl.ds(off, 16)]
            en = jnp.exp(jnp.minimum(z, 0.0))
            val = jnp.where(z > 0.0, z, _SELU_A * (en - 1.0))
            hidb[pl.ds(off, 16)] = _SELU_L * val
            return c
        lax.fori_loop(0, CH // 16, fb, 0, unroll=2)
        pltpu.sync_copy(hidb, hid_hbm.at[n])
        return carry
    lax.fori_loop(n0, n1, node_body, 0)


@functools.partial(
    pl.kernel,
    out_type=jax.ShapeDtypeStruct((NN, CH), jnp.float32),
    mesh=plsc.VectorSubcoreMesh(core_axis_name="c", subcore_axis_name="s"),
    compiler_params=pltpu.CompilerParams(needs_layout_passes=False),
    scratch_types=[
        pltpu.VMEM((HC,), jnp.float32),
        pltpu.VMEM((CH,), jnp.float32),
        pltpu.VMEM((HC,), jnp.float32),
        pltpu.VMEM((2 * G, HC), jnp.float32),
        pltpu.VMEM((2 * G,), jnp.int32),
        pltpu.VMEM((16,), jnp.int32),
        pltpu.VMEM((16,), jnp.int32),
        pltpu.VMEM((HC,), jnp.float32),
        pltpu.VMEM((16,), jnp.float32),
        pltpu.VMEM((16,), jnp.float32),
        pltpu.VMEM((CH,), jnp.float32),
        pltpu.SemaphoreType.DMA,
    ],
)
def _sc_gat(xl_hbm, xr_hbm, idx_hbm, meta_hbm, bounds_hbm, att_hbm,
            gatb_hbm, hid_hbm, *scratch):
    _sc_gat_body(xl_hbm, xr_hbm, idx_hbm, meta_hbm, bounds_hbm, att_hbm,
                 gatb_hbm, hid_hbm, *scratch)


def kernel(edge_index, mesh_feat, x, conv_feat, W_emb, b_emb, Wq, bq, Wk, bk,
           Wv, bv, Wo, bo, ln1_g, ln1_b, W_ff1, b_ff1, W_ff2, b_ff2, ln2_g,
           ln2_b, W_tout, b_tout, W_lin, b_lin, W_l, b_l, W_r, b_r, att,
           gat_b, W_out2, b_out2):
    i32 = jnp.int32

    mf3 = mesh_feat.reshape(BSZ, SEQ, 4)
    x3 = x.reshape(BSZ, SEQ, 7)
    hidden = _t1(mf3, x3, W_emb, b_emb, Wq, bq, Wk, bk, Wv, bv, Wo, bo,
                 ln1_g, ln1_b, W_ff1, b_ff1, W_ff2, b_ff2, ln2_g, ln2_b,
                 W_tout, b_tout, W_lin, b_lin).reshape(NN, CH)
    phi = mesh_feat[:, 3:4]

    ar = jnp.arange(NN, dtype=edge_index.dtype)
    src_all = jnp.concatenate([edge_index[0], ar])
    dst_all = jnp.concatenate([edge_index[1], ar])
    order = jnp.argsort(dst_all)
    ssrc = src_all[order].astype(i32)
    sdst = dst_all[order].astype(i32)
    deg = jnp.zeros((NN,), i32).at[dst_all].add(1)
    pdeg = ((deg + (G - 1)) // G) * G
    prow = jnp.concatenate([jnp.zeros((1,), i32), jnp.cumsum(pdeg)])
    rrow = jnp.concatenate([jnp.zeros((1,), i32), jnp.cumsum(deg)])
    pos = prow[sdst] + (jnp.arange(E1, dtype=i32) - rrow[sdst])
    idx_pad = jnp.zeros((E_PAD,), i32).at[pos].set(ssrc)
    meta = (jnp.zeros((NN, 16), i32)
            .at[:, 0].set(prow[:NN])
            .at[:, 1].set(pdeg // G)
            .at[:, 2].set(deg))
    cumch = prow // G
    total_ch = cumch[NN]
    targets = (jnp.arange(1, NW, dtype=i32) * total_ch) // NW
    bvals = jnp.searchsorted(cumch, targets).astype(i32)
    bounds = jnp.concatenate(
        [jnp.zeros((1,), i32), bvals, jnp.full((1,), NN, i32)])
    bounds2 = (jnp.zeros((NW, 16), i32)
               .at[:, 0].set(bounds[:NW])
               .at[:, 1].set(bounds[1:]))

    attf = att.reshape(HC)
    bl2, br2 = b_l.reshape(1, HC), b_r.reshape(1, HC)
    b2d = b_out2.reshape(1, 1)

    for _ in range(3):
        xin = jnp.concatenate([phi, hidden], axis=1)
        xl, xr = _t2(xin, W_l, bl2, W_r, br2)
        hidden = _sc_gat(xl, xr, idx_pad, meta, bounds2, attf, gat_b)
        phi = _k3(hidden, W_out2, b2d)
    return phi


# speedup vs baseline: 1.7239x; 1.1249x over previous
import functools

import jax
import jax.numpy as jnp
from jax import lax
from jax.experimental import pallas as pl
from jax.experimental.pallas import tpu as pltpu
from jax.experimental.pallas import tpu_sc as plsc

NN = 10000
BSZ = 10
SEQ = 1000
NH = 6
CH = 512
HC = NH * CH
G = 8
E_RAW = 160000
E1 = E_RAW + NN
E_PAD = E1 + (G - 1) * NN
NW = 32

_SELU_L = 1.0507009873554805
_SELU_A = 1.6732632423543772


def _t1_body(mf_ref, x_ref, Wemb, bemb, Wq, bq, Wk, bk, Wv, bv, Wo, bo,
             g1, b1, Wff1, bff1, Wff2, bff2, g2, b2, Wtout, btout,
             Wlin, blin, hid_ref):
    f32 = jnp.float32

    def ln(t, g, b):
        mu = t.mean(-1, keepdims=True)
        var = t.var(-1, keepdims=True)
        return (t - mu) / jnp.sqrt(var + 1e-5) * g[...] + b[...]

    mf = mf_ref[0]
    h = jnp.dot(mf, Wemb[...], preferred_element_type=f32) + bemb[...]
    q = jnp.dot(h, Wq[...], preferred_element_type=f32) + bq[...]
    k = jnp.dot(h, Wk[...], preferred_element_type=f32) + bk[...]
    v = jnp.dot(h, Wv[...], preferred_element_type=f32) + bv[...]
    ao_heads = []
    for hh in range(4):
        sl = slice(hh * 16, (hh + 1) * 16)
        qs, ks, vs = q[:, sl], k[:, sl], v[:, sl]
        s = lax.dot_general(qs, ks, (((1,), (1,)), ((), ())),
                            preferred_element_type=f32) * 0.25
        s = jax.nn.softmax(s, axis=-1)
        ao_heads.append(jnp.dot(s, vs, preferred_element_type=f32))
    ao = jnp.concatenate(ao_heads, axis=-1)
    h = ln(h + jnp.dot(ao, Wo[...], preferred_element_type=f32) + bo[...],
           g1, b1)
    ff = jnp.dot(jax.nn.relu(jnp.dot(h, Wff1[...], preferred_element_type=f32)
                             + bff1[...]),
                 Wff2[...], preferred_element_type=f32) + bff2[...]
    h = ln(h + ff, g2, b2)
    feats = jnp.dot(h, Wtout[...], preferred_element_type=f32) + btout[...]
    xx = x_ref[0]
    fc = jnp.concatenate([xx[:, 2:], feats], axis=1)
    z = jnp.dot(fc, Wlin[...], preferred_element_type=f32) + blin[...]
    en = jnp.exp(jnp.minimum(z, 0.0))
    hid_ref[0] = _SELU_L * jnp.where(z > 0.0, z, _SELU_A * (en - 1.0))


def _t1(mf3, x3, Wemb, bemb, Wq, bq, Wk, bk, Wv, bv, Wo, bo, g1, b1,
        Wff1, bff1, Wff2, bff2, g2, b2, Wtout, btout, Wlin, blin):
    full = lambda *shape: pl.BlockSpec(shape, lambda b: (0,) * len(shape))
    return pl.pallas_call(
        _t1_body,
        grid=(BSZ,),
        in_specs=[
            pl.BlockSpec((1, SEQ, 4), lambda b: (b, 0, 0)),
            pl.BlockSpec((1, SEQ, 7), lambda b: (b, 0, 0)),
            full(4, 64), full(64), full(64, 64), full(64), full(64, 64),
            full(64), full(64, 64), full(64), full(64, 64), full(64),
            full(64), full(64), full(64, 256), full(256), full(256, 64),
            full(64), full(64), full(64), full(64, 16), full(16),
            full(21, 512), full(512),
        ],
        out_specs=pl.BlockSpec((1, SEQ, CH), lambda b: (b, 0, 0)),
        out_shape=jax.ShapeDtypeStruct((BSZ, SEQ, CH), jnp.float32),
    )(mf3, x3, Wemb, bemb, Wq, bq, Wk, bk, Wv, bv, Wo, bo, g1, b1,
      Wff1, bff1, Wff2, bff2, g2, b2, Wtout, btout, Wlin, blin)


_BMROW = 1000
_BCOL = 1024


def _t2_body(xin_ref, Wl_ref, bl_ref, Wr_ref, br_ref, xl_ref, xr_ref):
    f32 = jnp.float32
    h = xin_ref[...]
    xl_ref[...] = (jnp.dot(h, Wl_ref[...], preferred_element_type=f32)
                   + bl_ref[...])
    xr_ref[...] = (jnp.dot(h, Wr_ref[...], preferred_element_type=f32)
                   + br_ref[...])


def _t2(xin, W_l, bl2, W_r, br2):
    return pl.pallas_call(
        _t2_body,
        grid=(NN // _BMROW, HC // _BCOL),
        in_specs=[
            pl.BlockSpec((_BMROW, 513), lambda i, j: (i, 0)),
            pl.BlockSpec((513, _BCOL), lambda i, j: (0, j)),
            pl.BlockSpec((1, _BCOL), lambda i, j: (0, j)),
            pl.BlockSpec((513, _BCOL), lambda i, j: (0, j)),
            pl.BlockSpec((1, _BCOL), lambda i, j: (0, j)),
        ],
        out_specs=[
            pl.BlockSpec((_BMROW, _BCOL), lambda i, j: (i, j)),
            pl.BlockSpec((_BMROW, _BCOL), lambda i, j: (i, j)),
        ],
        out_shape=[
            jax.ShapeDtypeStruct((NN, HC), jnp.float32),
            jax.ShapeDtypeStruct((NN, HC), jnp.float32),
        ],
    )(xin, W_l, bl2, W_r, br2)


def _k3_body(hid_ref, w_ref, b_ref, phi_ref):
    phi_ref[...] = (jnp.dot(hid_ref[...], w_ref[...],
                            preferred_element_type=jnp.float32)
                    + b_ref[...])


def _k3(hidden, W_out2, b2d):
    return pl.pallas_call(
        _k3_body,
        grid=(NN // _BMROW,),
        in_specs=[
            pl.BlockSpec((_BMROW, CH), lambda i: (i, 0)),
            pl.BlockSpec((CH, 1), lambda i: (0, 0)),
            pl.BlockSpec((1, 1), lambda i: (0, 0)),
        ],
        out_specs=pl.BlockSpec((_BMROW, 1), lambda i: (i, 0)),
        out_shape=jax.ShapeDtypeStruct((NN, 1), jnp.float32),
    )(hidden, W_out2, b2d)


def _sc_gat_body(xl_hbm, xr_hbm, idx_hbm, meta_hbm, bounds_hbm, att_hbm,
                 gatb_hbm, hid_hbm,
                 att_v, gatb_v, xr_v, rows_v, idx_v, meta_v, bnd_v,
                 acc_v, mbuf, sbuf, hidb, sem):
    f32 = jnp.float32
    wid = lax.axis_index("s") * 2 + lax.axis_index("c")
    pltpu.sync_copy(att_hbm, att_v)
    pltpu.sync_copy(gatb_hbm, gatb_v)
    pltpu.sync_copy(bounds_hbm.at[wid], bnd_v)
    bvec = bnd_v[...]
    n0 = bvec[0]
    n1 = bvec[1]
    iota16 = lax.iota(jnp.int32, 16)

    neg_big = jnp.full((16,), -3e38, f32)

    def _stage_idx(j, slot):
        off_e = pl.multiple_of(start_ref[0] + j * G, G)
        dst = pl.multiple_of(slot * G, G)
        pltpu.sync_copy(idx_hbm.at[pl.ds(off_e, G)],
                        idx_v.at[pl.ds(dst, G)])

    def _gather(j, slot, sem):
        dst = pl.multiple_of(slot * G, G)
        return pltpu.make_async_copy(
            xl_hbm.at[idx_v.at[pl.ds(dst, G)]],
            rows_v.at[pl.ds(dst, G)], sem)

    start_ref = [None]

    def node_body(n, carry):
        pltpu.sync_copy(meta_hbm.at[n], meta_v)
        pltpu.sync_copy(xr_hbm.at[n], xr_v)
        mvec = meta_v[...]
        start = mvec[0]
        nch = mvec[1]
        deg = mvec[2]
        start_ref[0] = start
        mbuf[...] = neg_big

        _stage_idx(0, 0)
        _gather(0, 0, sem).start()

        @pl.when(nch > 1)
        def _():
            _stage_idx(1, 1)

        def chunk_body(j, c2):
            slot = j & 1
            _gather(j, slot, sem).wait()

            @pl.when(j + 1 < nch)
            def _():
                _gather(j + 1, 1 - slot, sem).start()

            @pl.when(j + 2 < nch)
            def _():
                _stage_idx(j + 2, slot)

            rbase = slot * G
            lvs = [None] * G
            for hh in range(NH):
                off0 = hh * CH

                def cb(ci, vas):
                    off = off0 + ci * 16
                    xrv = xr_v[pl.ds(off, 16)]
                    atv = att_v[pl.ds(off, 16)]
                    out = []
                    for e in range(G):
                        z = rows_v[rbase + e, pl.ds(off, 16)] + xrv
                        z = jnp.maximum(z, z * 0.2)
                        out.append(vas[e] + z * atv)
                    return tuple(out)
                vas = lax.fori_loop(0, CH // 16, cb,
                                    (jnp.zeros((16,), f32),) * G, unroll=2)
                for e in range(G):
                    lg = jnp.sum(vas[e])
                    lv = lvs[e]
                    lvs[e] = (jnp.where(iota16 == hh, lg, neg_big) if lv is None
                              else jnp.where(iota16 == hh, lg, lv))
            base_e = j * G
            for e in range(G):
                lvs[e] = jnp.where(base_e + e < deg, lvs[e], neg_big)
            def _tree(vals, op):
                vals = list(vals)
                while len(vals) > 1:
                    vals = [op(vals[i], vals[i + 1])
                            for i in range(0, len(vals), 2)]
                return vals[0]

            mc = _tree(lvs, jnp.maximum)
            m = mbuf[...]
            mn = jnp.maximum(m, mc)
            scv = jnp.exp(m - mn)
            mbuf[...] = mn
            ws = [jnp.exp(lv - mn) for lv in lvs]
            wsum = _tree(ws, jnp.add)
            sbuf[...] = sbuf[...] * scv + wsum
            sch = tuple(scv[hh] for hh in range(NH))
            weh = [[ws[e][hh] for hh in range(NH)] for e in range(G)]
            for hh in range(NH):
                off0 = hh * CH
                sc_h = sch[hh]
                w_h = [weh[e][hh] for e in range(G)]

                def ab(ci, c4):
                    off = off0 + ci * 16
                    terms = [rows_v[rbase + e, pl.ds(off, 16)] * w_h[e]
                             for e in range(G)]
                    while len(terms) > 1:
                        terms = [terms[i] + terms[i + 1]
                                 for i in range(0, len(terms), 2)]
                    acc_v[pl.ds(off, 16)] = (acc_v[pl.ds(off, 16)] * sc_h
                                             + terms[0])
                    return c4
                lax.fori_loop(0, CH // 16, ab, 0, unroll=2)
            return c2
        lax.fori_loop(0, nch, chunk_body, 0)

        coef = 1.0 / (6.0 * (sbuf[...] + 1e-16))
        cs = tuple(coef[hh] for hh in range(NH))

        def fb(ci, c):
            off = ci * 16
            hv = acc_v[pl.ds(off, 16)] * cs[0]
            for hh in range(1, NH):
                hv = hv + acc_v[pl.ds(hh * CH + off, 16)] * cs[hh]
            z = hv + gatb_v[pl.ds(off, 16)]
            en = jnp.exp(jnp.minimum(z, 0.0))
            val = jnp.where(z > 0.0, z, _SELU_A * (en - 1.0))
            hidb[pl.ds(off, 16)] = _SELU_L * val
            return c
        lax.fori_loop(0, CH // 16, fb, 0, unroll=2)
        pltpu.sync_copy(hidb, hid_hbm.at[n])
        return carry
    lax.fori_loop(n0, n1, node_body, 0)


@functools.partial(
    pl.kernel,
    out_type=jax.ShapeDtypeStruct((NN, CH), jnp.float32),
    mesh=plsc.VectorSubcoreMesh(core_axis_name="c", subcore_axis_name="s"),
    compiler_params=pltpu.CompilerParams(needs_layout_passes=False),
    scratch_types=[
        pltpu.VMEM((HC,), jnp.float32),
        pltpu.VMEM((CH,), jnp.float32),
        pltpu.VMEM((HC,), jnp.float32),
        pltpu.VMEM((2 * G, HC), jnp.float32),
        pltpu.VMEM((2 * G,), jnp.int32),
        pltpu.VMEM((16,), jnp.int32),
        pltpu.VMEM((16,), jnp.int32),
        pltpu.VMEM((HC,), jnp.float32),
        pltpu.VMEM((16,), jnp.float32),
        pltpu.VMEM((16,), jnp.float32),
        pltpu.VMEM((CH,), jnp.float32),
        pltpu.SemaphoreType.DMA,
    ],
)
def _sc_gat(xl_hbm, xr_hbm, idx_hbm, meta_hbm, bounds_hbm, att_hbm,
            gatb_hbm, hid_hbm, *scratch):
    _sc_gat_body(xl_hbm, xr_hbm, idx_hbm, meta_hbm, bounds_hbm, att_hbm,
                 gatb_hbm, hid_hbm, *scratch)


def kernel(edge_index, mesh_feat, x, conv_feat, W_emb, b_emb, Wq, bq, Wk, bk,
           Wv, bv, Wo, bo, ln1_g, ln1_b, W_ff1, b_ff1, W_ff2, b_ff2, ln2_g,
           ln2_b, W_tout, b_tout, W_lin, b_lin, W_l, b_l, W_r, b_r, att,
           gat_b, W_out2, b_out2):
    i32 = jnp.int32

    mf3 = mesh_feat.reshape(BSZ, SEQ, 4)
    x3 = x.reshape(BSZ, SEQ, 7)
    hidden = _t1(mf3, x3, W_emb, b_emb, Wq, bq, Wk, bk, Wv, bv, Wo, bo,
                 ln1_g, ln1_b, W_ff1, b_ff1, W_ff2, b_ff2, ln2_g, ln2_b,
                 W_tout, b_tout, W_lin, b_lin).reshape(NN, CH)
    phi = mesh_feat[:, 3:4]

    ar = jnp.arange(NN, dtype=edge_index.dtype)
    src_all = jnp.concatenate([edge_index[0], ar])
    dst_all = jnp.concatenate([edge_index[1], ar])
    order = jnp.argsort(dst_all)
    ssrc = src_all[order].astype(i32)
    sdst = dst_all[order].astype(i32)
    deg = jnp.zeros((NN,), i32).at[dst_all].add(1)
    pdeg = ((deg + (G - 1)) // G) * G
    prow = jnp.concatenate([jnp.zeros((1,), i32), jnp.cumsum(pdeg)])
    rrow = jnp.concatenate([jnp.zeros((1,), i32), jnp.cumsum(deg)])
    pos = prow[sdst] + (jnp.arange(E1, dtype=i32) - rrow[sdst])
    idx_pad = jnp.zeros((E_PAD,), i32).at[pos].set(ssrc)
    meta = (jnp.zeros((NN, 16), i32)
            .at[:, 0].set(prow[:NN])
            .at[:, 1].set(pdeg // G)
            .at[:, 2].set(deg))
    cumch = prow // G
    total_ch = cumch[NN]
    targets = (jnp.arange(1, NW, dtype=i32) * total_ch) // NW
    bvals = jnp.searchsorted(cumch, targets).astype(i32)
    bounds = jnp.concatenate(
        [jnp.zeros((1,), i32), bvals, jnp.full((1,), NN, i32)])
    bounds2 = (jnp.zeros((NW, 16), i32)
               .at[:, 0].set(bounds[:NW])
               .at[:, 1].set(bounds[1:]))

    attf = att.reshape(HC)
    bl2, br2 = b_l.reshape(1, HC), b_r.reshape(1, HC)
    b2d = b_out2.reshape(1, 1)

    for _ in range(3):
        xin = jnp.concatenate([phi, hidden], axis=1)
        xl, xr = _t2(xin, W_l, bl2, W_r, br2)
        hidden = _sc_gat(xl, xr, idx_pad, meta, bounds2, attf, gat_b)
        phi = _k3(hidden, W_out2, b2d)
    return phi
